# baseline (speedup 1.0000x reference)
import functools

import jax
import jax.numpy as jnp
from jax import lax
from jax.experimental import pallas as pl
from jax.experimental.pallas import tpu as pltpu
from jax.experimental.pallas import tpu_sc as plsc

N_USERS = 5000
N_MOVIES = 5000
N = N_USERS + N_MOVIES
E = 160000
D = 128
B = 16384

NC = 2
NS = 16
NW = NC * NS
CHUNK = 128
NCHUNKS = E // CHUNK
NPW = 40
E_PAD = NW * NPW * CHUNK
NPAD = 10240
ROWS_PT = NPAD // NS
QCHUNK = 128
QPER_W = B // NW


@functools.lru_cache(maxsize=None)
def _mesh():
    return plsc.VectorSubcoreMesh(core_axis_name="c", subcore_axis_name="s",
                                  num_cores=NC, num_subcores=NS)


def _zero_acc(zrows_hbm, rows, acc, r0):
    pltpu.sync_copy(zrows_hbm, rows)
    for j in range(ROWS_PT // CHUNK):
        pltpu.sync_copy(rows, acc.at[pl.ds(r0 + j * CHUNK, CHUNK), :])


def _copy_out(acc, rows, out_hbm, cid, r0):
    for j in range(ROWS_PT // CHUNK):
        rr = r0 + j * CHUNK
        pltpu.sync_copy(acc.at[pl.ds(rr, CHUNK), :], rows)
        pltpu.sync_copy(rows, out_hbm.at[cid, pl.ds(rr, CHUNK), :])


def _sc_agg_body(src_hbm, dst_hbm, h_hbm, zrows_hbm,
                 agg_hbm,
                 sidx, didx, rows_a, rows_b, acc, sem_a, sem_b):
    cid = lax.axis_index("c")
    sid = lax.axis_index("s")
    wid = sid * NC + cid
    r0 = sid * ROWS_PT

    _zero_acc(zrows_hbm, rows_a, acc, r0)

    pltpu.sync_copy(src_hbm.at[pl.ds(wid * NPW, NPW), :], sidx)
    pltpu.sync_copy(dst_hbm.at[pl.ds(wid * NPW, NPW), :], didx)
    plsc.subcore_barrier()

    def start(i, rows, sem):
        return pltpu.async_copy(h_hbm.at[sidx.at[i]], rows, sem)

    def drain(i, rows):
        pltpu.sync_copy(rows, acc.at[didx.at[i]], add=True)

    def body(j, carry):
        ca = start(2 * j, rows_a, sem_a)
        cb = start(2 * j + 1, rows_b, sem_b)
        ca.wait()
        drain(2 * j, rows_a)
        cb.wait()
        drain(2 * j + 1, rows_b)
        return carry

    lax.fori_loop(0, NPW // 2, body, 0)
    plsc.subcore_barrier()
    _copy_out(acc, rows_a, agg_hbm, cid, r0)


@functools.lru_cache(maxsize=None)
def _make_sc_agg():
    return pl.kernel(
        _sc_agg_body,
        out_type=jax.ShapeDtypeStruct((NC, NPAD, D), jnp.float32),
        mesh=_mesh(),
        scratch_types=(
            pltpu.VMEM((NPW, CHUNK), jnp.int32),
            pltpu.VMEM((NPW, CHUNK), jnp.int32),
            pltpu.VMEM((CHUNK, D), jnp.float32),
            pltpu.VMEM((CHUNK, D), jnp.float32),
            pltpu.VMEM_SHARED((NPAD, D), jnp.float32),
            pltpu.SemaphoreType.DMA,
            pltpu.SemaphoreType.DMA,
        ),
        name="sc_agg",
    )


def _sc_agg(*args):
    return _make_sc_agg()(*args)


def _sc_deg_body(dst_hbm, zrows_hbm, ones_hbm,
                 deg_hbm,
                 didx, rows, ones_v, dacc):
    cid = lax.axis_index("c")
    sid = lax.axis_index("s")
    wid = sid * NC + cid
    r0 = sid * ROWS_PT

    _zero_acc(zrows_hbm, rows, dacc, r0)
    pltpu.sync_copy(ones_hbm, ones_v)
    pltpu.sync_copy(dst_hbm.at[pl.ds(wid * NPW, NPW), :], didx)
    plsc.subcore_barrier()

    def body(i, carry):
        pltpu.sync_copy(ones_v, dacc.at[didx.at[i]], add=True)
        return carry

    lax.fori_loop(0, NPW, body, 0)
    plsc.subcore_barrier()
    _copy_out(dacc, rows, deg_hbm, cid, r0)


@functools.lru_cache(maxsize=None)
def _make_sc_deg():
    return pl.kernel(
        _sc_deg_body,
        out_type=jax.ShapeDtypeStruct((NC, NPAD, D), jnp.float32),
        mesh=_mesh(),
        scratch_types=(
            pltpu.VMEM((NPW, CHUNK), jnp.int32),
            pltpu.VMEM((CHUNK, D), jnp.float32),
            pltpu.VMEM((CHUNK, D), jnp.float32),
            pltpu.VMEM_SHARED((NPAD, D), jnp.float32),
        ),
        name="sc_deg",
    )


def _sc_deg(*args):
    return _make_sc_deg()(*args)


def _sc_qgather_body(uidx_hbm, midx_hbm, h_hbm, u_hbm, m_hbm,
                     idx_v, rows, sem):
    cid = lax.axis_index("c")
    sid = lax.axis_index("s")
    wid = sid * NC + cid
    base = wid * QPER_W
    for side_hbm, out_hbm in ((uidx_hbm, u_hbm), (midx_hbm, m_hbm)):
        for j in range(QPER_W // QCHUNK):
            off = base + j * QCHUNK
            pltpu.sync_copy(side_hbm.at[pl.ds(off, QCHUNK)], idx_v)
            pltpu.async_copy(h_hbm.at[idx_v], rows, sem).wait()
            pltpu.sync_copy(rows, out_hbm.at[pl.ds(off, QCHUNK), :])


@functools.lru_cache(maxsize=None)
def _make_sc_qgather():
    return pl.kernel(
        _sc_qgather_body,
        out_type=(jax.ShapeDtypeStruct((B, D), jnp.float32),
                  jax.ShapeDtypeStruct((B, D), jnp.float32)),
        mesh=_mesh(),
        scratch_types=(
            pltpu.VMEM((QCHUNK,), jnp.int32),
            pltpu.VMEM((QCHUNK, D), jnp.float32),
            pltpu.SemaphoreType.DMA,
        ),
        name="sc_qgather",
    )


def _sc_qgather(*args):
    return _make_sc_qgather()(*args)


def _bn_body(ut_ref, mt_ref, g_ref, be_ref, out_ref):
    g = g_ref[...]
    be = be_ref[...]
    for ref, off in ((ut_ref, 0), (mt_ref, N_USERS)):
        x = ref[...]
        mu = jnp.mean(x, axis=0, keepdims=True)
        xc = x - mu
        var = jnp.mean(xc * xc, axis=0, keepdims=True)
        y = g * xc * lax.rsqrt(var + 1e-5) + be
        out_ref[off:off + N_USERS, :] = y


def _bn(ut, mt, g, be):
    return pl.pallas_call(
        _bn_body,
        out_shape=jax.ShapeDtypeStruct((N, D), jnp.float32),
    )(ut, mt, g.reshape(1, D), be.reshape(1, D))


LR = 2000


def _layer_body(h_ref, ag_ref, dg_ref, h0_ref, ws_ref, wn_ref, b_ref, out_ref):
    deg = dg_ref[0, :, 0:1] + dg_ref[1, :, 0:1]
    scale = 1.0 / jnp.maximum(deg, 1.0)
    hn = (ag_ref[0] + ag_ref[1]) * scale
    acc = lax.dot_general(h_ref[...], ws_ref[...],
                          (((1,), (1,)), ((), ())),
                          preferred_element_type=jnp.float32)
    acc = acc + lax.dot_general(hn, wn_ref[...],
                                (((1,), (1,)), ((), ())),
                                preferred_element_type=jnp.float32)
    out = jnp.maximum(acc + b_ref[...], 0.0)
    out_ref[...] = jnp.maximum(out + h0_ref[...], 0.0)


def _layer(h, agg, deg, h0, ws, wn, b):
    grid = (N // LR,)
    return pl.pallas_call(
        _layer_body,
        grid=grid,
        in_specs=[
            pl.BlockSpec((LR, D), lambda i: (i, 0)),
            pl.BlockSpec((NC, LR, D), lambda i: (0, i, 0)),
            pl.BlockSpec((NC, LR, D), lambda i: (0, i, 0)),
            pl.BlockSpec((LR, D), lambda i: (i, 0)),
            pl.BlockSpec((D, D), lambda i: (0, 0)),
            pl.BlockSpec((D, D), lambda i: (0, 0)),
            pl.BlockSpec((1, D), lambda i: (0, 0)),
        ],
        out_specs=pl.BlockSpec((LR, D), lambda i: (i, 0)),
        out_shape=jax.ShapeDtypeStruct((N, D), jnp.float32),
    )(h, agg, deg, h0, ws, wn, b.reshape(1, D))


MR = 2048


def _mlp_body(u_ref, m_ref, w1_ref, b1_ref, w2_ref, b2_ref, w3_ref, b3_ref,
              out_ref):
    w1 = w1_ref[...]
    z1 = lax.dot_general(u_ref[...], w1[:, 0:D],
                         (((1,), (1,)), ((), ())),
                         preferred_element_type=jnp.float32)
    z1 = z1 + lax.dot_general(m_ref[...], w1[:, D:2 * D],
                              (((1,), (1,)), ((), ())),
                              preferred_element_type=jnp.float32)
    z1 = jnp.maximum(z1 + b1_ref[...], 0.0)
    z2 = lax.dot_general(z1, w2_ref[...],
                         (((1,), (1,)), ((), ())),
                         preferred_element_type=jnp.float32)
    z2 = jnp.maximum(z2 + b2_ref[...], 0.0)
    pred = jnp.sum(z2 * w3_ref[...], axis=1) + b3_ref[0, 0]
    out_ref[...] = pred


def _mlp(u, m, w1, b1, w2, b2, w3, b3):
    grid = (B // MR,)
    return pl.pallas_call(
        _mlp_body,
        grid=grid,
        in_specs=[
            pl.BlockSpec((MR, D), lambda i: (i, 0)),
            pl.BlockSpec((MR, D), lambda i: (i, 0)),
            pl.BlockSpec((D, 2 * D), lambda i: (0, 0)),
            pl.BlockSpec((1, D), lambda i: (0, 0)),
            pl.BlockSpec((D // 2, D), lambda i: (0, 0)),
            pl.BlockSpec((1, D // 2), lambda i: (0, 0)),
            pl.BlockSpec((1, D // 2), lambda i: (0, 0)),
            pl.BlockSpec((1, 1), lambda i: (0, 0)),
        ],
        out_specs=pl.BlockSpec((MR,), lambda i: (i,)),
        out_shape=jax.ShapeDtypeStruct((B,), jnp.float32),
    )(u, m, w1, b1.reshape(1, D), w2, b2.reshape(1, D // 2), w3,
      b3.reshape(1, 1))


def kernel(edge_index, user_indices, movie_indices, user_table, movie_table,
           bn_gamma, bn_beta, Wself0, Wneigh0, b0, Wself1, Wneigh1, b1,
           Wself2, Wneigh2, b2, Wp1, bp1, Wp2, bp2, Wp3, bp3):
    src = edge_index[0].astype(jnp.int32)
    dst = edge_index[1].astype(jnp.int32)
    uidx = user_indices.astype(jnp.int32)
    midx = (movie_indices + N_USERS).astype(jnp.int32)

    src2 = jnp.concatenate(
        [src, jnp.zeros((E_PAD - E,), jnp.int32)]).reshape(-1, CHUNK)
    dst2 = jnp.concatenate(
        [dst, N + jnp.arange(E_PAD - E, dtype=jnp.int32) % (NPAD - N)]
    ).reshape(-1, CHUNK)

    zrows = jnp.zeros((CHUNK, D), jnp.float32)
    ones128 = jnp.ones((CHUNK, D), jnp.float32)

    h0 = _bn(user_table, movie_table, bn_gamma, bn_beta)

    deg = _sc_deg(dst2, zrows, ones128)
    agg = _sc_agg(src2, dst2, h0, zrows)
    h = _layer(h0, agg, deg, h0, Wself0, Wneigh0, b0)
    for ws, wn, bb in ((Wself1, Wneigh1, b1), (Wself2, Wneigh2, b2)):
        agg = _sc_agg(src2, dst2, h, zrows)
        h = _layer(h, agg, deg, h0, ws, wn, bb)

    u, m = _sc_qgather(uidx, midx, h)
    return _mlp(u, m, Wp1, bp1, Wp2, bp2, Wp3, bp3)


# speedup vs baseline: 2.7852x; 2.7852x over previous
import functools

import jax
import jax.numpy as jnp
from jax import lax
from jax.experimental import pallas as pl
from jax.experimental.pallas import tpu as pltpu
from jax.experimental.pallas import tpu_sc as plsc

N_USERS = 5000
N_MOVIES = 5000
N = N_USERS + N_MOVIES
E = 160000
D = 128
B = 16384

NC = 2
NS = 16
NW = NC * NS
CHUNK = 128
NCHUNKS = E // CHUNK
NPW = 40
E_PAD = NW * NPW * CHUNK
NPAD = 10240
ROWS_PT = NPAD // NS
QCHUNK = 128
QPER_W = B // NW


@functools.lru_cache(maxsize=None)
def _mesh():
    return plsc.VectorSubcoreMesh(core_axis_name="c", subcore_axis_name="s",
                                  num_cores=NC, num_subcores=NS)


def _zero_acc(zrows_hbm, rows, acc, r0):
    pltpu.sync_copy(zrows_hbm, rows)
    for j in range(ROWS_PT // CHUNK):
        pltpu.sync_copy(rows, acc.at[pl.ds(r0 + j * CHUNK, CHUNK), :])


def _copy_out(acc, rows, out_hbm, cid, r0):
    for j in range(ROWS_PT // CHUNK):
        rr = r0 + j * CHUNK
        pltpu.sync_copy(acc.at[pl.ds(rr, CHUNK), :], rows)
        pltpu.sync_copy(rows, out_hbm.at[cid, pl.ds(rr, CHUNK), :])


def _sc_agg_body(src_hbm, dst_hbm, h_hbm, zrows_hbm,
                 agg_hbm,
                 sidx, didx, rows_a, rows_b, acc, sem_a, sem_b):
    cid = lax.axis_index("c")
    sid = lax.axis_index("s")
    wid = sid * NC + cid
    r0 = sid * ROWS_PT

    _zero_acc(zrows_hbm, rows_a, acc, r0)

    pltpu.sync_copy(src_hbm.at[pl.ds(wid * NPW, NPW), :], sidx)
    pltpu.sync_copy(dst_hbm.at[pl.ds(wid * NPW, NPW), :], didx)
    plsc.subcore_barrier()

    def start(i, rows, sem):
        return pltpu.async_copy(h_hbm.at[sidx.at[i]], rows, sem)

    def drain(i, rows):
        pltpu.sync_copy(rows, acc.at[didx.at[i]], add=True)

    start(0, rows_a, sem_a)

    def body(j, carry):
        start(2 * j + 1, rows_b, sem_b)
        pltpu.make_async_copy(h_hbm.at[sidx.at[2 * j]], rows_a, sem_a).wait()
        drain(2 * j, rows_a)

        @pl.when(j < NPW // 2 - 1)
        def _():
            start(2 * j + 2, rows_a, sem_a)

        pltpu.make_async_copy(h_hbm.at[sidx.at[2 * j + 1]], rows_b,
                              sem_b).wait()
        drain(2 * j + 1, rows_b)
        return carry

    lax.fori_loop(0, NPW // 2, body, 0)
    plsc.subcore_barrier()
    _copy_out(acc, rows_a, agg_hbm, cid, r0)


@functools.lru_cache(maxsize=None)
def _make_sc_agg():
    return pl.kernel(
        _sc_agg_body,
        out_type=jax.ShapeDtypeStruct((NC, NPAD, D), jnp.float32),
        mesh=_mesh(),
        scratch_types=(
            pltpu.VMEM((NPW, CHUNK), jnp.int32),
            pltpu.VMEM((NPW, CHUNK), jnp.int32),
            pltpu.VMEM((CHUNK, D), jnp.float32),
            pltpu.VMEM((CHUNK, D), jnp.float32),
            pltpu.VMEM_SHARED((NPAD, D), jnp.float32),
            pltpu.SemaphoreType.DMA,
            pltpu.SemaphoreType.DMA,
        ),
        name="sc_agg",
    )


def _sc_agg(*args):
    return _make_sc_agg()(*args)


def _sc_deg_body(dst_hbm, zrows_hbm, ones_hbm,
                 deg_hbm,
                 didx, rows, ones_v, dacc):
    cid = lax.axis_index("c")
    sid = lax.axis_index("s")
    wid = sid * NC + cid
    r0 = sid * ROWS_PT

    _zero_acc(zrows_hbm, rows, dacc, r0)
    pltpu.sync_copy(ones_hbm, ones_v)
    pltpu.sync_copy(dst_hbm.at[pl.ds(wid * NPW, NPW), :], didx)
    plsc.subcore_barrier()

    def body(i, carry):
        pltpu.sync_copy(ones_v, dacc.at[didx.at[i]], add=True)
        return carry

    lax.fori_loop(0, NPW, body, 0)
    plsc.subcore_barrier()
    _copy_out(dacc, rows, deg_hbm, cid, r0)


@functools.lru_cache(maxsize=None)
def _make_sc_deg():
    return pl.kernel(
        _sc_deg_body,
        out_type=jax.ShapeDtypeStruct((NC, NPAD, D), jnp.float32),
        mesh=_mesh(),
        scratch_types=(
            pltpu.VMEM((NPW, CHUNK), jnp.int32),
            pltpu.VMEM((CHUNK, D), jnp.float32),
            pltpu.VMEM((CHUNK, D), jnp.float32),
            pltpu.VMEM_SHARED((NPAD, D), jnp.float32),
        ),
        name="sc_deg",
    )


def _sc_deg(*args):
    return _make_sc_deg()(*args)


def _sc_qgather_body(uidx_hbm, midx_hbm, h_hbm, u_hbm, m_hbm,
                     idx_v, rows, sem):
    cid = lax.axis_index("c")
    sid = lax.axis_index("s")
    wid = sid * NC + cid
    base = wid * QPER_W
    for side_hbm, out_hbm in ((uidx_hbm, u_hbm), (midx_hbm, m_hbm)):
        for j in range(QPER_W // QCHUNK):
            off = base + j * QCHUNK
            pltpu.sync_copy(side_hbm.at[pl.ds(off, QCHUNK)], idx_v)
            pltpu.async_copy(h_hbm.at[idx_v], rows, sem).wait()
            pltpu.sync_copy(rows, out_hbm.at[pl.ds(off, QCHUNK), :])


@functools.lru_cache(maxsize=None)
def _make_sc_qgather():
    return pl.kernel(
        _sc_qgather_body,
        out_type=(jax.ShapeDtypeStruct((B, D), jnp.float32),
                  jax.ShapeDtypeStruct((B, D), jnp.float32)),
        mesh=_mesh(),
        scratch_types=(
            pltpu.VMEM((QCHUNK,), jnp.int32),
            pltpu.VMEM((QCHUNK, D), jnp.float32),
            pltpu.SemaphoreType.DMA,
        ),
        name="sc_qgather",
    )


def _sc_qgather(*args):
    return _make_sc_qgather()(*args)


def _bn_body(ut_ref, mt_ref, g_ref, be_ref, out_ref):
    g = g_ref[...]
    be = be_ref[...]
    for ref, off in ((ut_ref, 0), (mt_ref, N_USERS)):
        x = ref[...]
        mu = jnp.mean(x, axis=0, keepdims=True)
        xc = x - mu
        var = jnp.mean(xc * xc, axis=0, keepdims=True)
        y = g * xc * lax.rsqrt(var + 1e-5) + be
        out_ref[off:off + N_USERS, :] = y


def _bn(ut, mt, g, be):
    return pl.pallas_call(
        _bn_body,
        out_shape=jax.ShapeDtypeStruct((N, D), jnp.float32),
    )(ut, mt, g.reshape(1, D), be.reshape(1, D))


LR = 2000


def _layer_body(h_ref, ag_ref, dg_ref, h0_ref, ws_ref, wn_ref, b_ref, out_ref):
    deg = dg_ref[0, :, 0:1] + dg_ref[1, :, 0:1]
    scale = 1.0 / jnp.maximum(deg, 1.0)
    hn = (ag_ref[0] + ag_ref[1]) * scale
    acc = lax.dot_general(h_ref[...], ws_ref[...],
                          (((1,), (1,)), ((), ())),
                          preferred_element_type=jnp.float32)
    acc = acc + lax.dot_general(hn, wn_ref[...],
                                (((1,), (1,)), ((), ())),
                                preferred_element_type=jnp.float32)
    out = jnp.maximum(acc + b_ref[...], 0.0)
    out_ref[...] = jnp.maximum(out + h0_ref[...], 0.0)


def _layer(h, agg, deg, h0, ws, wn, b):
    grid = (N // LR,)
    return pl.pallas_call(
        _layer_body,
        grid=grid,
        in_specs=[
            pl.BlockSpec((LR, D), lambda i: (i, 0)),
            pl.BlockSpec((NC, LR, D), lambda i: (0, i, 0)),
            pl.BlockSpec((NC, LR, D), lambda i: (0, i, 0)),
            pl.BlockSpec((LR, D), lambda i: (i, 0)),
            pl.BlockSpec((D, D), lambda i: (0, 0)),
            pl.BlockSpec((D, D), lambda i: (0, 0)),
            pl.BlockSpec((1, D), lambda i: (0, 0)),
        ],
        out_specs=pl.BlockSpec((LR, D), lambda i: (i, 0)),
        out_shape=jax.ShapeDtypeStruct((N, D), jnp.float32),
    )(h, agg, deg, h0, ws, wn, b.reshape(1, D))


MR = 2048


def _mlp_body(u_ref, m_ref, w1_ref, b1_ref, w2_ref, b2_ref, w3_ref, b3_ref,
              out_ref):
    w1 = w1_ref[...]
    z1 = lax.dot_general(u_ref[...], w1[:, 0:D],
                         (((1,), (1,)), ((), ())),
                         preferred_element_type=jnp.float32)
    z1 = z1 + lax.dot_general(m_ref[...], w1[:, D:2 * D],
                              (((1,), (1,)), ((), ())),
                              preferred_element_type=jnp.float32)
    z1 = jnp.maximum(z1 + b1_ref[...], 0.0)
    z2 = lax.dot_general(z1, w2_ref[...],
                         (((1,), (1,)), ((), ())),
                         preferred_element_type=jnp.float32)
    z2 = jnp.maximum(z2 + b2_ref[...], 0.0)
    pred = jnp.sum(z2 * w3_ref[...], axis=1) + b3_ref[0, 0]
    out_ref[...] = pred


def _mlp(u, m, w1, b1, w2, b2, w3, b3):
    grid = (B // MR,)
    return pl.pallas_call(
        _mlp_body,
        grid=grid,
        in_specs=[
            pl.BlockSpec((MR, D), lambda i: (i, 0)),
            pl.BlockSpec((MR, D), lambda i: (i, 0)),
            pl.BlockSpec((D, 2 * D), lambda i: (0, 0)),
            pl.BlockSpec((1, D), lambda i: (0, 0)),
            pl.BlockSpec((D // 2, D), lambda i: (0, 0)),
            pl.BlockSpec((1, D // 2), lambda i: (0, 0)),
            pl.BlockSpec((1, D // 2), lambda i: (0, 0)),
            pl.BlockSpec((1, 1), lambda i: (0, 0)),
        ],
        out_specs=pl.BlockSpec((MR,), lambda i: (i,)),
        out_shape=jax.ShapeDtypeStruct((B,), jnp.float32),
    )(u, m, w1, b1.reshape(1, D), w2, b2.reshape(1, D // 2), w3,
      b3.reshape(1, 1))


def kernel(edge_index, user_indices, movie_indices, user_table, movie_table,
           bn_gamma, bn_beta, Wself0, Wneigh0, b0, Wself1, Wneigh1, b1,
           Wself2, Wneigh2, b2, Wp1, bp1, Wp2, bp2, Wp3, bp3):
    src = edge_index[0].astype(jnp.int32)
    dst = edge_index[1].astype(jnp.int32)
    uidx = user_indices.astype(jnp.int32)
    midx = (movie_indices + N_USERS).astype(jnp.int32)

    src2 = jnp.concatenate(
        [src, jnp.arange(E_PAD - E, dtype=jnp.int32) % N]
    ).reshape(-1, CHUNK)
    dst2 = jnp.concatenate(
        [dst, N + jnp.arange(E_PAD - E, dtype=jnp.int32) % (NPAD - N)]
    ).reshape(-1, CHUNK)

    zrows = jnp.zeros((CHUNK, D), jnp.float32)
    ones128 = jnp.ones((CHUNK, D), jnp.float32)

    h0 = _bn(user_table, movie_table, bn_gamma, bn_beta)

    deg = _sc_deg(dst2, zrows, ones128)
    agg = _sc_agg(src2, dst2, h0, zrows)
    h = _layer(h0, agg, deg, h0, Wself0, Wneigh0, b0)
    for ws, wn, bb in ((Wself1, Wneigh1, b1), (Wself2, Wneigh2, b2)):
        agg = _sc_agg(src2, dst2, h, zrows)
        h = _layer(h, agg, deg, h0, ws, wn, bb)

    u, m = _sc_qgather(uidx, midx, h)
    return _mlp(u, m, Wp1, bp1, Wp2, bp2, Wp3, bp3)


# speedup vs baseline: 2.9391x; 1.0552x over previous
import functools

import jax
import jax.numpy as jnp
from jax import lax
from jax.experimental import pallas as pl
from jax.experimental.pallas import tpu as pltpu
from jax.experimental.pallas import tpu_sc as plsc

N_USERS = 5000
N_MOVIES = 5000
N = N_USERS + N_MOVIES
E = 160000
D = 128
B = 16384

NC = 2
NS = 16
NW = NC * NS
CHUNK = 128
NCHUNKS = E // CHUNK
NPW = 40
E_PAD = NW * NPW * CHUNK
NPAD = 10240
ROWS_PT = NPAD // NS
QCHUNK = 128
QPER_W = B // NW
DEGW = 32


@functools.lru_cache(maxsize=None)
def _mesh():
    return plsc.VectorSubcoreMesh(core_axis_name="c", subcore_axis_name="s",
                                  num_cores=NC, num_subcores=NS)


def _zero_acc(zrows_hbm, rows, acc, r0):
    pltpu.sync_copy(zrows_hbm, rows)
    for j in range(ROWS_PT // CHUNK):
        pltpu.sync_copy(rows, acc.at[pl.ds(r0 + j * CHUNK, CHUNK), :])


def _copy_out(acc, rows, out_hbm, cid, r0):
    for j in range(ROWS_PT // CHUNK):
        rr = r0 + j * CHUNK
        pltpu.sync_copy(acc.at[pl.ds(rr, CHUNK), :], rows)
        pltpu.sync_copy(rows, out_hbm.at[cid, pl.ds(rr, CHUNK), :])


def _sc_agg_body(src_hbm, dst_hbm, h_hbm, zrows_hbm,
                 agg_hbm,
                 sidx, didx, rows_a, rows_b, acc, sem_a, sem_b):
    cid = lax.axis_index("c")
    sid = lax.axis_index("s")
    wid = sid * NC + cid
    r0 = sid * ROWS_PT

    _zero_acc(zrows_hbm, rows_a, acc, r0)

    pltpu.sync_copy(src_hbm.at[pl.ds(wid * NPW, NPW), :], sidx)
    pltpu.sync_copy(dst_hbm.at[pl.ds(wid * NPW, NPW), :], didx)
    plsc.subcore_barrier()

    def start(i, rows, sem):
        return pltpu.async_copy(h_hbm.at[sidx.at[i]], rows, sem)

    def drain(i, rows):
        pltpu.sync_copy(rows, acc.at[didx.at[i]], add=True)

    start(0, rows_a, sem_a)

    def body(j, carry):
        start(2 * j + 1, rows_b, sem_b)
        pltpu.make_async_copy(h_hbm.at[sidx.at[2 * j]], rows_a, sem_a).wait()
        drain(2 * j, rows_a)

        @pl.when(j < NPW // 2 - 1)
        def _():
            start(2 * j + 2, rows_a, sem_a)

        pltpu.make_async_copy(h_hbm.at[sidx.at[2 * j + 1]], rows_b,
                              sem_b).wait()
        drain(2 * j + 1, rows_b)
        return carry

    lax.fori_loop(0, NPW // 2, body, 0)
    plsc.subcore_barrier()
    _copy_out(acc, rows_a, agg_hbm, cid, r0)


@functools.lru_cache(maxsize=None)
def _make_sc_agg():
    return pl.kernel(
        _sc_agg_body,
        out_type=jax.ShapeDtypeStruct((NC, NPAD, D), jnp.float32),
        mesh=_mesh(),
        scratch_types=(
            pltpu.VMEM((NPW, CHUNK), jnp.int32),
            pltpu.VMEM((NPW, CHUNK), jnp.int32),
            pltpu.VMEM((CHUNK, D), jnp.float32),
            pltpu.VMEM((CHUNK, D), jnp.float32),
            pltpu.VMEM_SHARED((NPAD, D), jnp.float32),
            pltpu.SemaphoreType.DMA,
            pltpu.SemaphoreType.DMA,
        ),
        name="sc_agg",
    )


def _sc_agg(*args):
    return _make_sc_agg()(*args)


def _sc_deg_body(dst_hbm, zrows_hbm, ones_hbm,
                 deg_hbm,
                 didx, rows, ones_v, dacc):
    cid = lax.axis_index("c")
    sid = lax.axis_index("s")
    wid = sid * NC + cid
    r0 = sid * ROWS_PT

    _zero_acc(zrows_hbm, rows, dacc, r0)
    pltpu.sync_copy(ones_hbm, ones_v)
    pltpu.sync_copy(dst_hbm.at[pl.ds(wid * NPW, NPW), :], didx)
    plsc.subcore_barrier()

    def body(i, carry):
        pltpu.sync_copy(ones_v, dacc.at[didx.at[i]], add=True)
        return carry

    lax.fori_loop(0, NPW, body, 0)
    plsc.subcore_barrier()
    _copy_out(dacc, rows, deg_hbm, cid, r0)


@functools.lru_cache(maxsize=None)
def _make_sc_deg():
    return pl.kernel(
        _sc_deg_body,
        out_type=jax.ShapeDtypeStruct((NC, NPAD, DEGW), jnp.float32),
        mesh=_mesh(),
        scratch_types=(
            pltpu.VMEM((NPW, CHUNK), jnp.int32),
            pltpu.VMEM((CHUNK, DEGW), jnp.float32),
            pltpu.VMEM((CHUNK, DEGW), jnp.float32),
            pltpu.VMEM_SHARED((NPAD, DEGW), jnp.float32),
        ),
        name="sc_deg",
    )


def _sc_deg(*args):
    return _make_sc_deg()(*args)


def _sc_qgather_body(uidx_hbm, midx_hbm, h_hbm, u_hbm, m_hbm,
                     idx_v, rows, sem):
    cid = lax.axis_index("c")
    sid = lax.axis_index("s")
    wid = sid * NC + cid
    base = wid * QPER_W
    for side_hbm, out_hbm in ((uidx_hbm, u_hbm), (midx_hbm, m_hbm)):
        for j in range(QPER_W // QCHUNK):
            off = base + j * QCHUNK
            pltpu.sync_copy(side_hbm.at[pl.ds(off, QCHUNK)], idx_v)
            pltpu.async_copy(h_hbm.at[idx_v], rows, sem).wait()
            pltpu.sync_copy(rows, out_hbm.at[pl.ds(off, QCHUNK), :])


@functools.lru_cache(maxsize=None)
def _make_sc_qgather():
    return pl.kernel(
        _sc_qgather_body,
        out_type=(jax.ShapeDtypeStruct((B, D), jnp.float32),
                  jax.ShapeDtypeStruct((B, D), jnp.float32)),
        mesh=_mesh(),
        scratch_types=(
            pltpu.VMEM((QCHUNK,), jnp.int32),
            pltpu.VMEM((QCHUNK, D), jnp.float32),
            pltpu.SemaphoreType.DMA,
        ),
        name="sc_qgather",
    )


def _sc_qgather(*args):
    return _make_sc_qgather()(*args)


def _bn_body(ut_ref, mt_ref, g_ref, be_ref, out_ref):
    g = g_ref[...]
    be = be_ref[...]
    for ref, off in ((ut_ref, 0), (mt_ref, N_USERS)):
        x = ref[...]
        mu = jnp.mean(x, axis=0, keepdims=True)
        xc = x - mu
        var = jnp.mean(xc * xc, axis=0, keepdims=True)
        y = g * xc * lax.rsqrt(var + 1e-5) + be
        out_ref[off:off + N_USERS, :] = y


def _bn(ut, mt, g, be):
    return pl.pallas_call(
        _bn_body,
        out_shape=jax.ShapeDtypeStruct((N, D), jnp.float32),
    )(ut, mt, g.reshape(1, D), be.reshape(1, D))


LR = 2000


def _layer_body(h_ref, ag_ref, dg_ref, h0_ref, ws_ref, wn_ref, b_ref, out_ref):
    deg = dg_ref[0, :, 0:1] + dg_ref[1, :, 0:1]
    scale = 1.0 / jnp.maximum(deg, 1.0)
    hn = (ag_ref[0] + ag_ref[1]) * scale
    acc = lax.dot_general(h_ref[...], ws_ref[...],
                          (((1,), (1,)), ((), ())),
                          preferred_element_type=jnp.float32)
    acc = acc + lax.dot_general(hn, wn_ref[...],
                                (((1,), (1,)), ((), ())),
                                preferred_element_type=jnp.float32)
    out = jnp.maximum(acc + b_ref[...], 0.0)
    out_ref[...] = jnp.maximum(out + h0_ref[...], 0.0)


def _layer(h, agg, deg, h0, ws, wn, b):
    grid = (N // LR,)
    return pl.pallas_call(
        _layer_body,
        grid=grid,
        in_specs=[
            pl.BlockSpec((LR, D), lambda i: (i, 0)),
            pl.BlockSpec((NC, LR, D), lambda i: (0, i, 0)),
            pl.BlockSpec((NC, LR, DEGW), lambda i: (0, i, 0)),
            pl.BlockSpec((LR, D), lambda i: (i, 0)),
            pl.BlockSpec((D, D), lambda i: (0, 0)),
            pl.BlockSpec((D, D), lambda i: (0, 0)),
            pl.BlockSpec((1, D), lambda i: (0, 0)),
        ],
        out_specs=pl.BlockSpec((LR, D), lambda i: (i, 0)),
        out_shape=jax.ShapeDtypeStruct((N, D), jnp.float32),
    )(h, agg, deg, h0, ws, wn, b.reshape(1, D))


MR = 2048


def _mlp_body(u_ref, m_ref, w1_ref, b1_ref, w2_ref, b2_ref, w3_ref, b3_ref,
              out_ref):
    w1 = w1_ref[...]
    z1 = lax.dot_general(u_ref[...], w1[:, 0:D],
                         (((1,), (1,)), ((), ())),
                         preferred_element_type=jnp.float32)
    z1 = z1 + lax.dot_general(m_ref[...], w1[:, D:2 * D],
                              (((1,), (1,)), ((), ())),
                              preferred_element_type=jnp.float32)
    z1 = jnp.maximum(z1 + b1_ref[...], 0.0)
    z2 = lax.dot_general(z1, w2_ref[...],
                         (((1,), (1,)), ((), ())),
                         preferred_element_type=jnp.float32)
    z2 = jnp.maximum(z2 + b2_ref[...], 0.0)
    pred = jnp.sum(z2 * w3_ref[...], axis=1) + b3_ref[0, 0]
    out_ref[...] = pred


def _mlp(u, m, w1, b1, w2, b2, w3, b3):
    grid = (B // MR,)
    return pl.pallas_call(
        _mlp_body,
        grid=grid,
        in_specs=[
            pl.BlockSpec((MR, D), lambda i: (i, 0)),
            pl.BlockSpec((MR, D), lambda i: (i, 0)),
            pl.BlockSpec((D, 2 * D), lambda i: (0, 0)),
            pl.BlockSpec((1, D), lambda i: (0, 0)),
            pl.BlockSpec((D // 2, D), lambda i: (0, 0)),
            pl.BlockSpec((1, D // 2), lambda i: (0, 0)),
            pl.BlockSpec((1, D // 2), lambda i: (0, 0)),
            pl.BlockSpec((1, 1), lambda i: (0, 0)),
        ],
        out_specs=pl.BlockSpec((MR,), lambda i: (i,)),
        out_shape=jax.ShapeDtypeStruct((B,), jnp.float32),
    )(u, m, w1, b1.reshape(1, D), w2, b2.reshape(1, D // 2), w3,
      b3.reshape(1, 1))


def kernel(edge_index, user_indices, movie_indices, user_table, movie_table,
           bn_gamma, bn_beta, Wself0, Wneigh0, b0, Wself1, Wneigh1, b1,
           Wself2, Wneigh2, b2, Wp1, bp1, Wp2, bp2, Wp3, bp3):
    src = edge_index[0].astype(jnp.int32)
    dst = edge_index[1].astype(jnp.int32)
    uidx = user_indices.astype(jnp.int32)
    midx = (movie_indices + N_USERS).astype(jnp.int32)

    src2 = jnp.concatenate(
        [src, jnp.arange(E_PAD - E, dtype=jnp.int32) % N]
    ).reshape(-1, CHUNK)
    dst2 = jnp.concatenate(
        [dst, N + jnp.arange(E_PAD - E, dtype=jnp.int32) % (NPAD - N)]
    ).reshape(-1, CHUNK)

    zrows = jnp.zeros((CHUNK, D), jnp.float32)

    h0 = _bn(user_table, movie_table, bn_gamma, bn_beta)

    zdeg = jnp.zeros((CHUNK, DEGW), jnp.float32)
    onesdeg = jnp.ones((CHUNK, DEGW), jnp.float32)
    deg = _sc_deg(dst2, zdeg, onesdeg)
    agg = _sc_agg(src2, dst2, h0, zrows)
    h = _layer(h0, agg, deg, h0, Wself0, Wneigh0, b0)
    for ws, wn, bb in ((Wself1, Wneigh1, b1), (Wself2, Wneigh2, b2)):
        agg = _sc_agg(src2, dst2, h, zrows)
        h = _layer(h, agg, deg, h0, ws, wn, bb)

    u, m = _sc_qgather(uidx, midx, h)
    return _mlp(u, m, Wp1, bp1, Wp2, bp2, Wp3, bp3)
